# Initial kernel scaffold; baseline (speedup 1.0000x reference)
import jax
import jax.numpy as jnp
from jax.experimental import pallas as pl


def kernel(x, pos, batch, edge_index, edge_attr, Wsp, Wroot, bconv, W1, b1, g1, be1, W2, b2, g2, be2, W3, b3, g3, be3):
    raise NotImplementedError("write your pallas kernel here")


# speedup vs baseline: 14.8224x; 14.8224x over previous
import functools

import jax
import jax.numpy as jnp
from jax import lax
from jax.experimental import pallas as pl
from jax.experimental.pallas import tpu as pltpu
from jax.experimental.pallas import tpu_sc as plsc

N = 10000
E = 320000
KS = 5
NS_SAMPLES = 5000
KNN = 3

NC = 2
NSUB = 16
NW = NC * NSUB
LANES = 16

ROWW = 32
CNT_COL = 25
EPW = E // NW
EBATCH = 80
NGRP = EBATCH // LANES

GATHER_D = 48
NS_PAD = 5120
ROWS_PW = NS_PAD // NW

FPS_R = 80
FPS_C = 128

QBLK = 200


def _conv_scatter_body(x_hbm, src_hbm, dst_hbm, ea_hbm, zero_hbm, out_hbm,
                       x_v, src_v, dst_v, ea_v, vals_v, idx_v, b_sh):
    c = lax.axis_index("c")
    s = lax.axis_index("s")
    wid = c * NSUB + s
    ebase = wid * EPW

    zchunk = N // NSUB
    pltpu.sync_copy(zero_hbm.at[pl.ds(0, zchunk)], b_sh.at[pl.ds(s * zchunk, zchunk)])

    pltpu.sync_copy(x_hbm, x_v)
    pltpu.sync_copy(src_hbm.at[pl.ds(ebase, EPW)], src_v)
    pltpu.sync_copy(dst_hbm.at[pl.ds(ebase, EPW)], dst_v)
    pltpu.sync_copy(ea_hbm.at[pl.ds(2 * ebase, 2 * EPW)], ea_v)
    plsc.subcore_barrier()

    lanes = lax.iota(jnp.int32, LANES)
    ones16 = jnp.full((LANES,), 1.0, jnp.float32)
    zeros16 = jnp.zeros((LANES,), jnp.float32)

    @pl.loop(0, EPW // EBATCH)
    def _batch(b):
        @pl.loop(0, EBATCH)
        def _z(r):
            vals_v[r, pl.ds(0, LANES)] = zeros16
            vals_v[r, pl.ds(LANES, LANES)] = zeros16

        @pl.loop(0, NGRP)
        def _grp(g):
            off = b * EBATCH + g * LANES
            src16 = src_v[pl.ds(off, LANES)]
            dst16 = dst_v[pl.ds(off, LANES)]
            eidx = (off + lanes) * 2
            u = plsc.load_gather(ea_v, [eidx])
            v = plsc.load_gather(ea_v, [eidx + 1])
            xs = plsc.load_gather(x_v, [src16])
            vu = u * (KS - 1.0)
            vv = v * (KS - 1.0)
            bui = jnp.clip(vu.astype(jnp.int32), 0, KS - 2)
            bvi = jnp.clip(vv.astype(jnp.int32), 0, KS - 2)
            bu = bui.astype(jnp.float32)
            bv = bvi.astype(jnp.float32)
            fu = vu - bu
            fv = vv - bv
            row = g * LANES + lanes
            for i0 in (0, 1):
                for i1 in (0, 1):
                    w0 = fu if i0 else (1.0 - fu)
                    w1 = fv if i1 else (1.0 - fv)
                    val = w0 * w1 * xs
                    kk = (bui + i0) + KS * (bvi + i1)
                    plsc.addupdate_scatter(vals_v, [row, kk], val)
            plsc.addupdate_scatter(vals_v, [row, jnp.full((LANES,), CNT_COL, jnp.int32)], ones16)
            idx_v[pl.ds(g * LANES, LANES)] = dst16

        pltpu.sync_copy(vals_v, b_sh.at[idx_v], add=True)

    plsc.subcore_barrier()
    pltpu.sync_copy(b_sh.at[pl.ds(s * zchunk, zchunk)],
                    out_hbm.at[c, pl.ds(s * zchunk, zchunk)])


def _conv_scatter(x, src, dst, edge_attr_flat, zeros_rows):
    mesh = plsc.VectorSubcoreMesh(core_axis_name="c", subcore_axis_name="s")
    kfn = pl.kernel(
        _conv_scatter_body,
        out_type=jax.ShapeDtypeStruct((NC, N, ROWW), jnp.float32),
        mesh=mesh,
        scratch_types=[
            pltpu.VMEM((N,), jnp.float32),
            pltpu.VMEM((EPW,), jnp.int32),
            pltpu.VMEM((EPW,), jnp.int32),
            pltpu.VMEM((2 * EPW,), jnp.float32),
            pltpu.VMEM((EBATCH, ROWW), jnp.float32),
            pltpu.VMEM((EBATCH,), jnp.int32),
            pltpu.VMEM_SHARED((N, ROWW), jnp.float32),
        ],
        compiler_params=pltpu.CompilerParams(
            use_tc_tiling_on_sc=False, needs_layout_passes=False),
    )
    return kfn(x, src, dst, edge_attr_flat, zeros_rows)


def _combine_body(bext_ref, x_ref, wsp_ref, wroot_ref, bconv_ref, h_ref):
    b = bext_ref[0] + bext_ref[1]
    basis = b[:, :KS * KS]
    cnt = b[:, CNT_COL:CNT_COL + 1]
    num = jnp.dot(basis, wsp_ref[...], preferred_element_type=jnp.float32,
                  precision=lax.Precision.HIGHEST)
    aggr = num / jnp.maximum(cnt, 1.0)
    h = aggr + x_ref[...] * wroot_ref[...] + bconv_ref[...]
    h_ref[...] = jnp.where(h > 0.0, h, jnp.exp(h) - 1.0)


def _combine(bext, x2, wsp25, wroot, bconv2):
    return pl.pallas_call(
        _combine_body,
        out_shape=jax.ShapeDtypeStruct((N, 32), jnp.float32),
    )(bext, x2, wsp25, wroot, bconv2)


def _fps_body(px_ref, py_ref, idx_ref):
    px = px_ref[...]
    py = py_ref[...]
    lin = lax.broadcasted_iota(jnp.int32, (FPS_R, FPS_C), 0) * FPS_C + \
        lax.broadcasted_iota(jnp.int32, (FPS_R, FPS_C), 1)
    valid = lin < N
    sx = px_ref[0, 0]
    sy = py_ref[0, 0]
    dx = px - sx
    dy = py - sy
    d0 = dx * dx + dy * dy
    dists = jnp.where(valid, d0, -1.0)
    idx_ref[0] = 0

    def body(i, dists):
        m = jnp.max(dists)
        sel = jnp.min(jnp.where(dists == m, lin, jnp.int32(2**30)))
        oh = lin == sel
        nx = jnp.sum(jnp.where(oh, px, 0.0))
        ny = jnp.sum(jnp.where(oh, py, 0.0))
        ddx = px - nx
        ddy = py - ny
        d = ddx * ddx + ddy * ddy
        idx_ref[i] = sel
        return jnp.minimum(dists, d)

    lax.fori_loop(1, NS_SAMPLES, body, dists)


def _fps(px, py):
    return pl.pallas_call(
        _fps_body,
        out_shape=jax.ShapeDtypeStruct((NS_SAMPLES,), jnp.int32),
        out_specs=pl.BlockSpec(memory_space=pltpu.SMEM),
    )(px, py)


def _gather_body(tab_hbm, idx_hbm, out_hbm, idx_v, rows_v, sem):
    c = lax.axis_index("c")
    s = lax.axis_index("s")
    wid = s * NC + c
    base = wid * ROWS_PW
    pltpu.sync_copy(idx_hbm.at[pl.ds(base, ROWS_PW)], idx_v)
    pltpu.async_copy(tab_hbm.at[idx_v], rows_v, sem).wait()
    pltpu.sync_copy(rows_v, out_hbm.at[pl.ds(base, ROWS_PW)])


def _sc_gather(table, idx_pad):
    mesh = plsc.VectorSubcoreMesh(core_axis_name="c", subcore_axis_name="s")
    kfn = pl.kernel(
        _gather_body,
        out_type=jax.ShapeDtypeStruct((NS_PAD, GATHER_D), jnp.float32),
        mesh=mesh,
        scratch_types=[
            pltpu.VMEM((ROWS_PW,), jnp.int32),
            pltpu.VMEM((ROWS_PW, GATHER_D), jnp.float32),
            pltpu.SemaphoreType.DMA,
        ],
        compiler_params=pltpu.CompilerParams(
            use_tc_tiling_on_sc=False, needs_layout_passes=False),
    )
    return kfn(table, idx_pad)


def _knn_body(qx_ref, qy_ref, pdx_ref, pdy_ref, xd_ref, xi_ref):
    qx = qx_ref[...]
    qy = qy_ref[...]
    pdx = pdx_ref[...]
    pdy = pdy_ref[...]
    nd2 = pdx * pdx + pdy * pdy
    q2 = qx * qx + qy * qy
    qxb = qx.astype(jnp.bfloat16).astype(jnp.float32)
    qyb = qy.astype(jnp.bfloat16).astype(jnp.float32)
    pdxb = pdx.astype(jnp.bfloat16).astype(jnp.float32)
    pdyb = pdy.astype(jnp.bfloat16).astype(jnp.float32)
    mm = qxb * pdxb + qyb * pdyb
    d2 = (q2 + nd2) - 2.0 * mm
    citer = lax.broadcasted_iota(jnp.int32, (QBLK, NS_SAMPLES), 1)
    big_i = jnp.int32(2**30)
    inf = jnp.float32(jnp.inf)
    S = jnp.zeros((QBLK, NS_SAMPLES), jnp.float32)
    ws = jnp.zeros((QBLK, 1), jnp.float32)
    for _ in range(KNN):
        m = jnp.min(d2, axis=1, keepdims=True)
        j = jnp.min(jnp.where(d2 == m, citer, big_i), axis=1, keepdims=True)
        w = 1.0 / jnp.maximum(m, 1e-16)
        oh = citer == j
        S = S + jnp.where(oh, w, 0.0)
        ws = ws + w
        d2 = jnp.where(oh, inf, d2)
    xi = jnp.dot(S, xd_ref[...], preferred_element_type=jnp.float32,
                 precision=lax.Precision.HIGHEST)
    xi_ref[...] = xi / ws


def _knn(qx, qy, pdx, pdy, xd):
    grid = N // QBLK
    return pl.pallas_call(
        _knn_body,
        grid=(grid,),
        in_specs=[
            pl.BlockSpec((QBLK, 1), lambda i: (i, 0)),
            pl.BlockSpec((QBLK, 1), lambda i: (i, 0)),
            pl.BlockSpec((1, NS_SAMPLES), lambda i: (0, 0)),
            pl.BlockSpec((1, NS_SAMPLES), lambda i: (0, 0)),
            pl.BlockSpec((NS_SAMPLES, 32), lambda i: (0, 0)),
        ],
        out_specs=pl.BlockSpec((QBLK, 32), lambda i: (i, 0)),
        out_shape=jax.ShapeDtypeStruct((N, 32), jnp.float32),
    )(qx, qy, pdx, pdy, xd)


def _bn(h, g, beta):
    mu = jnp.mean(h, axis=0, keepdims=True)
    var = jnp.mean((h - mu) ** 2, axis=0, keepdims=True)
    return g * (h - mu) / jnp.sqrt(var + 1e-5) + beta


def _mlp_body(xi_ref, x2_ref, w1a_ref, w1b_ref, b1_ref, g1_ref, be1_ref,
              w2_ref, b2_ref, g2_ref, be2_ref, w3_ref, b3_ref, g3_ref, be3_ref,
              out_ref):
    def bdot(a, b):
        return jnp.dot(a.astype(jnp.bfloat16), b.astype(jnp.bfloat16),
                       preferred_element_type=jnp.float32)

    xi = xi_ref[...]
    x2b = x2_ref[...].astype(jnp.bfloat16).astype(jnp.float32)
    w1bb = w1b_ref[...].astype(jnp.bfloat16).astype(jnp.float32)
    h = bdot(xi, w1a_ref[...]) + x2b * w1bb + b1_ref[...]
    h = jnp.maximum(h, 0.0)
    h = _bn(h, g1_ref[...], be1_ref[...])
    h = bdot(h, w2_ref[...]) + b2_ref[...]
    h = jnp.maximum(h, 0.0)
    h = _bn(h, g2_ref[...], be2_ref[...])
    h = bdot(h, w3_ref[...]) + b3_ref[...]
    h = jnp.maximum(h, 0.0)
    h = _bn(h, g3_ref[...], be3_ref[...])
    out_ref[...] = 1.0 / (1.0 + jnp.exp(-h))


def _mlp(xi, x2, w1a, w1b, b1, g1, be1, W2, b2, g2, be2, W3, b3, g3, be3):
    args = (xi, x2, w1a, w1b, b1, g1, be1, W2, b2, g2, be2, W3, b3, g3, be3)
    return pl.pallas_call(
        _mlp_body,
        out_shape=jax.ShapeDtypeStruct((N, 1), jnp.float32),
    )(*args)


def kernel(x, pos, batch, edge_index, edge_attr, Wsp, Wroot, bconv,
           W1, b1, g1, be1, W2, b2, g2, be2, W3, b3, g3, be3):
    x = x.astype(jnp.float32)
    ea_flat = edge_attr.reshape(-1)
    zeros_rows = jnp.zeros((N // NSUB, ROWW), jnp.float32)

    bext = _conv_scatter(x, edge_index[0], edge_index[1], ea_flat, zeros_rows)

    x2 = x[:, None]
    wsp25 = Wsp[:, 0, :]
    h = _combine(bext, x2, wsp25, Wroot, bconv[None, :])

    posx = pos[:, 0]
    posy = pos[:, 1]
    pad = FPS_R * FPS_C - N
    px = jnp.pad(posx, (0, pad)).reshape(FPS_R, FPS_C)
    py = jnp.pad(posy, (0, pad)).reshape(FPS_R, FPS_C)
    idx = _fps(px, py)

    table = jnp.concatenate(
        [h, pos, jnp.zeros((N, GATHER_D - 34), jnp.float32)], axis=1)
    idx_pad = jnp.concatenate(
        [idx, jnp.zeros((NS_PAD - NS_SAMPLES,), jnp.int32)])
    g = _sc_gather(table, idx_pad)

    xd = g[:NS_SAMPLES, :32]
    pdx = g[:NS_SAMPLES, 32].reshape(1, NS_SAMPLES)
    pdy = g[:NS_SAMPLES, 33].reshape(1, NS_SAMPLES)
    qx = posx[:, None]
    qy = posy[:, None]
    xi = _knn(qx, qy, pdx, pdy, xd)

    out = _mlp(xi, x2, W1[:32, :], W1[32:33, :], b1[None, :], g1[None, :],
               be1[None, :], W2, b2[None, :], g2[None, :], be2[None, :],
               W3, b3[None, :], g3[None, :], be3[None, :])
    return out


# speedup vs baseline: 21.1691x; 1.4282x over previous
import functools

import jax
import jax.numpy as jnp
from jax import lax
from jax.experimental import pallas as pl
from jax.experimental.pallas import tpu as pltpu
from jax.experimental.pallas import tpu_sc as plsc

N = 10000
E = 320000
KS = 5
NS_SAMPLES = 5000
KNN = 3

NC = 2
NSUB = 16
NW = NC * NSUB
LANES = 16

ROWW = 32
CNT_COL = 25
EPW = E // NW
EBATCH = 80
NGRP = EBATCH // LANES

GATHER_D = 48
NS_PAD = 5120
ROWS_PW = NS_PAD // NW

FPS_R = 80
FPS_C = 128

QBLK = 200


def _conv_scatter_body(x_hbm, src_hbm, dst_hbm, ea_hbm, zero_hbm, out_hbm,
                       x_v, src_v, dst_v, ea_v, vals_v, idx_v, b_sh):
    c = lax.axis_index("c")
    s = lax.axis_index("s")
    wid = c * NSUB + s
    ebase = wid * EPW

    zchunk = N // NSUB
    pltpu.sync_copy(zero_hbm.at[pl.ds(0, zchunk)], b_sh.at[pl.ds(s * zchunk, zchunk)])

    pltpu.sync_copy(x_hbm, x_v)
    pltpu.sync_copy(src_hbm.at[pl.ds(ebase, EPW)], src_v)
    pltpu.sync_copy(dst_hbm.at[pl.ds(ebase, EPW)], dst_v)
    pltpu.sync_copy(ea_hbm.at[pl.ds(2 * ebase, 2 * EPW)], ea_v)
    plsc.subcore_barrier()

    lanes = lax.iota(jnp.int32, LANES)
    ones16 = jnp.full((LANES,), 1.0, jnp.float32)
    zeros16 = jnp.zeros((LANES,), jnp.float32)

    @pl.loop(0, EPW // EBATCH)
    def _batch(b):
        @pl.loop(0, EBATCH)
        def _z(r):
            vals_v[r, pl.ds(0, LANES)] = zeros16
            vals_v[r, pl.ds(LANES, LANES)] = zeros16

        @pl.loop(0, NGRP)
        def _grp(g):
            off = b * EBATCH + g * LANES
            src16 = src_v[pl.ds(off, LANES)]
            dst16 = dst_v[pl.ds(off, LANES)]
            eidx = (off + lanes) * 2
            u = plsc.load_gather(ea_v, [eidx])
            v = plsc.load_gather(ea_v, [eidx + 1])
            xs = plsc.load_gather(x_v, [src16])
            vu = u * (KS - 1.0)
            vv = v * (KS - 1.0)
            bui = jnp.clip(vu.astype(jnp.int32), 0, KS - 2)
            bvi = jnp.clip(vv.astype(jnp.int32), 0, KS - 2)
            bu = bui.astype(jnp.float32)
            bv = bvi.astype(jnp.float32)
            fu = vu - bu
            fv = vv - bv
            row = g * LANES + lanes
            for i0 in (0, 1):
                for i1 in (0, 1):
                    w0 = fu if i0 else (1.0 - fu)
                    w1 = fv if i1 else (1.0 - fv)
                    val = w0 * w1 * xs
                    kk = (bui + i0) + KS * (bvi + i1)
                    plsc.addupdate_scatter(vals_v, [row, kk], val)
            plsc.addupdate_scatter(vals_v, [row, jnp.full((LANES,), CNT_COL, jnp.int32)], ones16)
            idx_v[pl.ds(g * LANES, LANES)] = dst16

        pltpu.sync_copy(vals_v, b_sh.at[idx_v], add=True)

    plsc.subcore_barrier()
    pltpu.sync_copy(b_sh.at[pl.ds(s * zchunk, zchunk)],
                    out_hbm.at[c, pl.ds(s * zchunk, zchunk)])


def _conv_scatter(x, src, dst, edge_attr_flat, zeros_rows):
    mesh = plsc.VectorSubcoreMesh(core_axis_name="c", subcore_axis_name="s")
    kfn = pl.kernel(
        _conv_scatter_body,
        out_type=jax.ShapeDtypeStruct((NC, N, ROWW), jnp.float32),
        mesh=mesh,
        scratch_types=[
            pltpu.VMEM((N,), jnp.float32),
            pltpu.VMEM((EPW,), jnp.int32),
            pltpu.VMEM((EPW,), jnp.int32),
            pltpu.VMEM((2 * EPW,), jnp.float32),
            pltpu.VMEM((EBATCH, ROWW), jnp.float32),
            pltpu.VMEM((EBATCH,), jnp.int32),
            pltpu.VMEM_SHARED((N, ROWW), jnp.float32),
        ],
        compiler_params=pltpu.CompilerParams(
            use_tc_tiling_on_sc=False, needs_layout_passes=False),
    )
    return kfn(x, src, dst, edge_attr_flat, zeros_rows)


def _combine_body(bext_ref, x_ref, wsp_ref, wroot_ref, bconv_ref, h_ref):
    b = bext_ref[0] + bext_ref[1]
    basis = b[:, :KS * KS]
    cnt = b[:, CNT_COL:CNT_COL + 1]
    num = jnp.dot(basis, wsp_ref[...], preferred_element_type=jnp.float32,
                  precision=lax.Precision.HIGHEST)
    aggr = num / jnp.maximum(cnt, 1.0)
    h = aggr + x_ref[...] * wroot_ref[...] + bconv_ref[...]
    h_ref[...] = jnp.where(h > 0.0, h, jnp.exp(h) - 1.0)


def _combine(bext, x2, wsp25, wroot, bconv2):
    return pl.pallas_call(
        _combine_body,
        out_shape=jax.ShapeDtypeStruct((N, 32), jnp.float32),
    )(bext, x2, wsp25, wroot, bconv2)


def _fps_body(px_ref, py_ref, pxs_ref, pys_ref, idx_ref):
    px = px_ref[...]
    py = py_ref[...]
    lin = lax.broadcasted_iota(jnp.int32, (FPS_R, FPS_C), 0) * FPS_C + \
        lax.broadcasted_iota(jnp.int32, (FPS_R, FPS_C), 1)
    valid = lin < N
    sx = pxs_ref[0]
    sy = pys_ref[0]
    dx = px - sx
    dy = py - sy
    d0 = dx * dx + dy * dy
    dists = jnp.where(valid, d0, -1.0)
    idx_ref[0] = jnp.int32(0)

    def body(i, dists):
        sel = jnp.argmax(dists).astype(jnp.int32)
        nx = pxs_ref[sel]
        ny = pys_ref[sel]
        ddx = px - nx
        ddy = py - ny
        d = ddx * ddx + ddy * ddy
        idx_ref[i] = sel
        return jnp.minimum(dists, d)

    lax.fori_loop(1, NS_SAMPLES, body, dists)


def _fps(px, py, pxs, pys):
    return pl.pallas_call(
        _fps_body,
        in_specs=[
            pl.BlockSpec(memory_space=pltpu.VMEM),
            pl.BlockSpec(memory_space=pltpu.VMEM),
            pl.BlockSpec(memory_space=pltpu.SMEM),
            pl.BlockSpec(memory_space=pltpu.SMEM),
        ],
        out_shape=jax.ShapeDtypeStruct((NS_SAMPLES,), jnp.int32),
        out_specs=pl.BlockSpec(memory_space=pltpu.SMEM),
    )(px, py, pxs, pys)


def _gather_body(tab_hbm, idx_hbm, out_hbm, idx_v, rows_v, sem):
    c = lax.axis_index("c")
    s = lax.axis_index("s")
    wid = s * NC + c
    base = wid * ROWS_PW
    pltpu.sync_copy(idx_hbm.at[pl.ds(base, ROWS_PW)], idx_v)
    pltpu.async_copy(tab_hbm.at[idx_v], rows_v, sem).wait()
    pltpu.sync_copy(rows_v, out_hbm.at[pl.ds(base, ROWS_PW)])


def _sc_gather(table, idx_pad):
    mesh = plsc.VectorSubcoreMesh(core_axis_name="c", subcore_axis_name="s")
    kfn = pl.kernel(
        _gather_body,
        out_type=jax.ShapeDtypeStruct((NS_PAD, GATHER_D), jnp.float32),
        mesh=mesh,
        scratch_types=[
            pltpu.VMEM((ROWS_PW,), jnp.int32),
            pltpu.VMEM((ROWS_PW, GATHER_D), jnp.float32),
            pltpu.SemaphoreType.DMA,
        ],
        compiler_params=pltpu.CompilerParams(
            use_tc_tiling_on_sc=False, needs_layout_passes=False),
    )
    return kfn(table, idx_pad)


def _knn_body(qx_ref, qy_ref, pdx_ref, pdy_ref, xd_ref, xi_ref):
    qx = qx_ref[...]
    qy = qy_ref[...]
    pdx = pdx_ref[...]
    pdy = pdy_ref[...]
    nd2 = pdx * pdx + pdy * pdy
    q2 = qx * qx + qy * qy
    qxb = qx.astype(jnp.bfloat16).astype(jnp.float32)
    qyb = qy.astype(jnp.bfloat16).astype(jnp.float32)
    pdxb = pdx.astype(jnp.bfloat16).astype(jnp.float32)
    pdyb = pdy.astype(jnp.bfloat16).astype(jnp.float32)
    mm = qxb * pdxb + qyb * pdyb
    d2 = (q2 + nd2) - 2.0 * mm
    citer = lax.broadcasted_iota(jnp.int32, (QBLK, NS_SAMPLES), 1)
    big_i = jnp.int32(2**30)
    inf = jnp.float32(jnp.inf)
    S = jnp.zeros((QBLK, NS_SAMPLES), jnp.float32)
    ws = jnp.zeros((QBLK, 1), jnp.float32)
    for _ in range(KNN):
        m = jnp.min(d2, axis=1, keepdims=True)
        j = jnp.min(jnp.where(d2 == m, citer, big_i), axis=1, keepdims=True)
        w = 1.0 / jnp.maximum(m, 1e-16)
        oh = citer == j
        S = S + jnp.where(oh, w, 0.0)
        ws = ws + w
        d2 = jnp.where(oh, inf, d2)
    xi = jnp.dot(S, xd_ref[...], preferred_element_type=jnp.float32,
                 precision=lax.Precision.HIGHEST)
    xi_ref[...] = xi / ws


def _knn(qx, qy, pdx, pdy, xd):
    grid = N // QBLK
    return pl.pallas_call(
        _knn_body,
        grid=(grid,),
        in_specs=[
            pl.BlockSpec((QBLK, 1), lambda i: (i, 0)),
            pl.BlockSpec((QBLK, 1), lambda i: (i, 0)),
            pl.BlockSpec((1, NS_SAMPLES), lambda i: (0, 0)),
            pl.BlockSpec((1, NS_SAMPLES), lambda i: (0, 0)),
            pl.BlockSpec((NS_SAMPLES, 32), lambda i: (0, 0)),
        ],
        out_specs=pl.BlockSpec((QBLK, 32), lambda i: (i, 0)),
        out_shape=jax.ShapeDtypeStruct((N, 32), jnp.float32),
    )(qx, qy, pdx, pdy, xd)


def _bn(h, g, beta):
    mu = jnp.mean(h, axis=0, keepdims=True)
    var = jnp.mean((h - mu) ** 2, axis=0, keepdims=True)
    return g * (h - mu) / jnp.sqrt(var + 1e-5) + beta


def _mlp_body(xi_ref, x2_ref, w1a_ref, w1b_ref, b1_ref, g1_ref, be1_ref,
              w2_ref, b2_ref, g2_ref, be2_ref, w3_ref, b3_ref, g3_ref, be3_ref,
              out_ref):
    def bdot(a, b):
        return jnp.dot(a.astype(jnp.bfloat16), b.astype(jnp.bfloat16),
                       preferred_element_type=jnp.float32)

    xi = xi_ref[...]
    x2b = x2_ref[...].astype(jnp.bfloat16).astype(jnp.float32)
    w1bb = w1b_ref[...].astype(jnp.bfloat16).astype(jnp.float32)
    h = bdot(xi, w1a_ref[...]) + x2b * w1bb + b1_ref[...]
    h = jnp.maximum(h, 0.0)
    h = _bn(h, g1_ref[...], be1_ref[...])
    h = bdot(h, w2_ref[...]) + b2_ref[...]
    h = jnp.maximum(h, 0.0)
    h = _bn(h, g2_ref[...], be2_ref[...])
    h = bdot(h, w3_ref[...]) + b3_ref[...]
    h = jnp.maximum(h, 0.0)
    h = _bn(h, g3_ref[...], be3_ref[...])
    out_ref[...] = 1.0 / (1.0 + jnp.exp(-h))


def _mlp(xi, x2, w1a, w1b, b1, g1, be1, W2, b2, g2, be2, W3, b3, g3, be3):
    args = (xi, x2, w1a, w1b, b1, g1, be1, W2, b2, g2, be2, W3, b3, g3, be3)
    return pl.pallas_call(
        _mlp_body,
        out_shape=jax.ShapeDtypeStruct((N, 1), jnp.float32),
    )(*args)


def kernel(x, pos, batch, edge_index, edge_attr, Wsp, Wroot, bconv,
           W1, b1, g1, be1, W2, b2, g2, be2, W3, b3, g3, be3):
    x = x.astype(jnp.float32)
    ea_flat = edge_attr.reshape(-1)
    zeros_rows = jnp.zeros((N // NSUB, ROWW), jnp.float32)

    bext = _conv_scatter(x, edge_index[0], edge_index[1], ea_flat, zeros_rows)

    x2 = x[:, None]
    wsp25 = Wsp[:, 0, :]
    h = _combine(bext, x2, wsp25, Wroot, bconv[None, :])

    posx = pos[:, 0]
    posy = pos[:, 1]
    pad = FPS_R * FPS_C - N
    px = jnp.pad(posx, (0, pad)).reshape(FPS_R, FPS_C)
    py = jnp.pad(posy, (0, pad)).reshape(FPS_R, FPS_C)
    idx = _fps(px, py, posx, posy)

    table = jnp.concatenate(
        [h, pos, jnp.zeros((N, GATHER_D - 34), jnp.float32)], axis=1)
    idx_pad = jnp.concatenate(
        [idx, jnp.zeros((NS_PAD - NS_SAMPLES,), jnp.int32)])
    g = _sc_gather(table, idx_pad)

    xd = g[:NS_SAMPLES, :32]
    pdx = g[:NS_SAMPLES, 32].reshape(1, NS_SAMPLES)
    pdy = g[:NS_SAMPLES, 33].reshape(1, NS_SAMPLES)
    qx = posx[:, None]
    qy = posy[:, None]
    xi = _knn(qx, qy, pdx, pdy, xd)

    out = _mlp(xi, x2, W1[:32, :], W1[32:33, :], b1[None, :], g1[None, :],
               be1[None, :], W2, b2[None, :], g2[None, :], be2[None, :],
               W3, b3[None, :], g3[None, :], be3[None, :])
    return out


# speedup vs baseline: 21.1910x; 1.0010x over previous
import functools

import jax
import jax.numpy as jnp
from jax import lax
from jax.experimental import pallas as pl
from jax.experimental.pallas import tpu as pltpu
from jax.experimental.pallas import tpu_sc as plsc

N = 10000
E = 320000
KS = 5
NS_SAMPLES = 5000
KNN = 3

NC = 2
NSUB = 16
NW = NC * NSUB
LANES = 16

ROWW = 32
CNT_COL = 25
EPW = E // NW
EBATCH = 80
NGRP = EBATCH // LANES

GATHER_D = 48
NS_PAD = 5120
ROWS_PW = NS_PAD // NW

FPS_R = 80
FPS_C = 128

QBLK = 400


def _conv_scatter_body(x_hbm, src_hbm, dst_hbm, ea_hbm, zero_hbm, out_hbm,
                       x_v, src_v, dst_v, ea_v, vals_v, idx_v, b_sh):
    c = lax.axis_index("c")
    s = lax.axis_index("s")
    wid = c * NSUB + s
    ebase = wid * EPW

    zchunk = N // NSUB
    pltpu.sync_copy(zero_hbm.at[pl.ds(0, zchunk)], b_sh.at[pl.ds(s * zchunk, zchunk)])

    pltpu.sync_copy(x_hbm, x_v)
    pltpu.sync_copy(src_hbm.at[pl.ds(ebase, EPW)], src_v)
    pltpu.sync_copy(dst_hbm.at[pl.ds(ebase, EPW)], dst_v)
    pltpu.sync_copy(ea_hbm.at[pl.ds(2 * ebase, 2 * EPW)], ea_v)
    plsc.subcore_barrier()

    lanes = lax.iota(jnp.int32, LANES)
    ones16 = jnp.full((LANES,), 1.0, jnp.float32)
    zeros16 = jnp.zeros((LANES,), jnp.float32)

    @pl.loop(0, EPW // EBATCH)
    def _batch(b):
        @pl.loop(0, EBATCH)
        def _z(r):
            vals_v[r, pl.ds(0, LANES)] = zeros16
            vals_v[r, pl.ds(LANES, LANES)] = zeros16

        @pl.loop(0, NGRP)
        def _grp(g):
            off = b * EBATCH + g * LANES
            src16 = src_v[pl.ds(off, LANES)]
            dst16 = dst_v[pl.ds(off, LANES)]
            eidx = (off + lanes) * 2
            u = plsc.load_gather(ea_v, [eidx])
            v = plsc.load_gather(ea_v, [eidx + 1])
            xs = plsc.load_gather(x_v, [src16])
            vu = u * (KS - 1.0)
            vv = v * (KS - 1.0)
            bui = jnp.clip(vu.astype(jnp.int32), 0, KS - 2)
            bvi = jnp.clip(vv.astype(jnp.int32), 0, KS - 2)
            bu = bui.astype(jnp.float32)
            bv = bvi.astype(jnp.float32)
            fu = vu - bu
            fv = vv - bv
            row = g * LANES + lanes
            for i0 in (0, 1):
                for i1 in (0, 1):
                    w0 = fu if i0 else (1.0 - fu)
                    w1 = fv if i1 else (1.0 - fv)
                    val = w0 * w1 * xs
                    kk = (bui + i0) + KS * (bvi + i1)
                    plsc.addupdate_scatter(vals_v, [row, kk], val)
            plsc.addupdate_scatter(vals_v, [row, jnp.full((LANES,), CNT_COL, jnp.int32)], ones16)
            idx_v[pl.ds(g * LANES, LANES)] = dst16

        pltpu.sync_copy(vals_v, b_sh.at[idx_v], add=True)

    plsc.subcore_barrier()
    pltpu.sync_copy(b_sh.at[pl.ds(s * zchunk, zchunk)],
                    out_hbm.at[c, pl.ds(s * zchunk, zchunk)])


def _conv_scatter(x, src, dst, edge_attr_flat, zeros_rows):
    mesh = plsc.VectorSubcoreMesh(core_axis_name="c", subcore_axis_name="s")
    kfn = pl.kernel(
        _conv_scatter_body,
        out_type=jax.ShapeDtypeStruct((NC, N, ROWW), jnp.float32),
        mesh=mesh,
        scratch_types=[
            pltpu.VMEM((N,), jnp.float32),
            pltpu.VMEM((EPW,), jnp.int32),
            pltpu.VMEM((EPW,), jnp.int32),
            pltpu.VMEM((2 * EPW,), jnp.float32),
            pltpu.VMEM((EBATCH, ROWW), jnp.float32),
            pltpu.VMEM((EBATCH,), jnp.int32),
            pltpu.VMEM_SHARED((N, ROWW), jnp.float32),
        ],
        compiler_params=pltpu.CompilerParams(
            use_tc_tiling_on_sc=False, needs_layout_passes=False),
    )
    return kfn(x, src, dst, edge_attr_flat, zeros_rows)


def _combine_body(bext_ref, x_ref, wsp_ref, wroot_ref, bconv_ref, h_ref):
    b = bext_ref[0] + bext_ref[1]
    basis = b[:, :KS * KS]
    cnt = b[:, CNT_COL:CNT_COL + 1]
    num = jnp.dot(basis, wsp_ref[...], preferred_element_type=jnp.float32,
                  precision=lax.Precision.HIGHEST)
    aggr = num / jnp.maximum(cnt, 1.0)
    h = aggr + x_ref[...] * wroot_ref[...] + bconv_ref[...]
    h_ref[...] = jnp.where(h > 0.0, h, jnp.exp(h) - 1.0)


def _combine(bext, x2, wsp25, wroot, bconv2):
    return pl.pallas_call(
        _combine_body,
        out_shape=jax.ShapeDtypeStruct((N, 32), jnp.float32),
    )(bext, x2, wsp25, wroot, bconv2)


def _fps_body(px_ref, py_ref, pxs_ref, pys_ref, idx_ref):
    px = px_ref[...]
    py = py_ref[...]
    lin = lax.broadcasted_iota(jnp.int32, (FPS_R, FPS_C), 0) * FPS_C + \
        lax.broadcasted_iota(jnp.int32, (FPS_R, FPS_C), 1)
    valid = lin < N
    sx = pxs_ref[0]
    sy = pys_ref[0]
    dx = px - sx
    dy = py - sy
    d0 = dx * dx + dy * dy
    dists0 = jnp.where(valid, d0, -1.0)
    idx_ref[0] = jnp.int32(0)

    def body(i, dists):
        sel = jnp.argmax(dists).astype(jnp.int32)
        nx = pxs_ref[sel]
        ny = pys_ref[sel]
        ddx = px - nx
        ddy = py - ny
        d = ddx * ddx + ddy * ddy
        idx_ref[i] = sel
        return jnp.minimum(dists, d)

    lax.fori_loop(1, NS_SAMPLES, body, dists0)


def _fps(px, py, pxs, pys):
    return pl.pallas_call(
        _fps_body,
        in_specs=[
            pl.BlockSpec(memory_space=pltpu.VMEM),
            pl.BlockSpec(memory_space=pltpu.VMEM),
            pl.BlockSpec(memory_space=pltpu.SMEM),
            pl.BlockSpec(memory_space=pltpu.SMEM),
        ],
        out_shape=jax.ShapeDtypeStruct((NS_SAMPLES,), jnp.int32),
        out_specs=pl.BlockSpec(memory_space=pltpu.SMEM),
    )(px, py, pxs, pys)


def _gather_body(tab_hbm, idx_hbm, out_hbm, idx_v, rows_v, sem):
    c = lax.axis_index("c")
    s = lax.axis_index("s")
    wid = s * NC + c
    base = wid * ROWS_PW
    pltpu.sync_copy(idx_hbm.at[pl.ds(base, ROWS_PW)], idx_v)
    pltpu.async_copy(tab_hbm.at[idx_v], rows_v, sem).wait()
    pltpu.sync_copy(rows_v, out_hbm.at[pl.ds(base, ROWS_PW)])


def _sc_gather(table, idx_pad):
    mesh = plsc.VectorSubcoreMesh(core_axis_name="c", subcore_axis_name="s")
    kfn = pl.kernel(
        _gather_body,
        out_type=jax.ShapeDtypeStruct((NS_PAD, GATHER_D), jnp.float32),
        mesh=mesh,
        scratch_types=[
            pltpu.VMEM((ROWS_PW,), jnp.int32),
            pltpu.VMEM((ROWS_PW, GATHER_D), jnp.float32),
            pltpu.SemaphoreType.DMA,
        ],
        compiler_params=pltpu.CompilerParams(
            use_tc_tiling_on_sc=False, needs_layout_passes=False),
    )
    return kfn(table, idx_pad)


def _knn_body(qx_ref, qy_ref, pdx_ref, pdy_ref, xd_ref, xi_ref):
    qx = qx_ref[...]
    qy = qy_ref[...]
    pdx = pdx_ref[...]
    pdy = pdy_ref[...]
    nd2 = pdx * pdx + pdy * pdy
    q2 = qx * qx + qy * qy
    qxb = qx.astype(jnp.bfloat16).astype(jnp.float32)
    qyb = qy.astype(jnp.bfloat16).astype(jnp.float32)
    pdxb = pdx.astype(jnp.bfloat16).astype(jnp.float32)
    pdyb = pdy.astype(jnp.bfloat16).astype(jnp.float32)
    mm = qxb * pdxb + qyb * pdyb
    d2 = (q2 + nd2) - 2.0 * mm
    citer = lax.broadcasted_iota(jnp.int32, (QBLK, NS_SAMPLES), 1)
    inf = jnp.float32(jnp.inf)
    S = jnp.zeros((QBLK, NS_SAMPLES), jnp.float32)
    ws = jnp.zeros((QBLK, 1), jnp.float32)
    for _ in range(KNN):
        m = jnp.min(d2, axis=1, keepdims=True)
        j = jnp.argmin(d2, axis=1).astype(jnp.int32)[:, None]
        w = 1.0 / jnp.maximum(m, 1e-16)
        oh = citer == j
        S = S + jnp.where(oh, w, 0.0)
        ws = ws + w
        d2 = jnp.where(oh, inf, d2)
    xi = jnp.dot(S, xd_ref[...], preferred_element_type=jnp.float32,
                 precision=lax.Precision.HIGHEST)
    xi_ref[...] = xi / ws


def _knn(qx, qy, pdx, pdy, xd):
    grid = N // QBLK
    return pl.pallas_call(
        _knn_body,
        grid=(grid,),
        in_specs=[
            pl.BlockSpec((QBLK, 1), lambda i: (i, 0)),
            pl.BlockSpec((QBLK, 1), lambda i: (i, 0)),
            pl.BlockSpec((1, NS_SAMPLES), lambda i: (0, 0)),
            pl.BlockSpec((1, NS_SAMPLES), lambda i: (0, 0)),
            pl.BlockSpec((NS_SAMPLES, 32), lambda i: (0, 0)),
        ],
        out_specs=pl.BlockSpec((QBLK, 32), lambda i: (i, 0)),
        out_shape=jax.ShapeDtypeStruct((N, 32), jnp.float32),
    )(qx, qy, pdx, pdy, xd)


def _bn(h, g, beta):
    mu = jnp.mean(h, axis=0, keepdims=True)
    var = jnp.mean((h - mu) ** 2, axis=0, keepdims=True)
    return g * (h - mu) / jnp.sqrt(var + 1e-5) + beta


def _mlp_body(xi_ref, x2_ref, w1a_ref, w1b_ref, b1_ref, g1_ref, be1_ref,
              w2_ref, b2_ref, g2_ref, be2_ref, w3_ref, b3_ref, g3_ref, be3_ref,
              out_ref):
    def bdot(a, b):
        return jnp.dot(a.astype(jnp.bfloat16), b.astype(jnp.bfloat16),
                       preferred_element_type=jnp.float32)

    xi = xi_ref[...]
    x2b = x2_ref[...].astype(jnp.bfloat16).astype(jnp.float32)
    w1bb = w1b_ref[...].astype(jnp.bfloat16).astype(jnp.float32)
    h = bdot(xi, w1a_ref[...]) + x2b * w1bb + b1_ref[...]
    h = jnp.maximum(h, 0.0)
    h = _bn(h, g1_ref[...], be1_ref[...])
    h = bdot(h, w2_ref[...]) + b2_ref[...]
    h = jnp.maximum(h, 0.0)
    h = _bn(h, g2_ref[...], be2_ref[...])
    h = bdot(h, w3_ref[...]) + b3_ref[...]
    h = jnp.maximum(h, 0.0)
    h = _bn(h, g3_ref[...], be3_ref[...])
    out_ref[...] = 1.0 / (1.0 + jnp.exp(-h))


def _mlp(xi, x2, w1a, w1b, b1, g1, be1, W2, b2, g2, be2, W3, b3, g3, be3):
    args = (xi, x2, w1a, w1b, b1, g1, be1, W2, b2, g2, be2, W3, b3, g3, be3)
    return pl.pallas_call(
        _mlp_body,
        out_shape=jax.ShapeDtypeStruct((N, 1), jnp.float32),
    )(*args)


def kernel(x, pos, batch, edge_index, edge_attr, Wsp, Wroot, bconv,
           W1, b1, g1, be1, W2, b2, g2, be2, W3, b3, g3, be3):
    x = x.astype(jnp.float32)
    ea_flat = edge_attr.reshape(-1)
    zeros_rows = jnp.zeros((N // NSUB, ROWW), jnp.float32)

    bext = _conv_scatter(x, edge_index[0], edge_index[1], ea_flat, zeros_rows)

    x2 = x[:, None]
    wsp25 = Wsp[:, 0, :]
    h = _combine(bext, x2, wsp25, Wroot, bconv[None, :])

    posx = pos[:, 0]
    posy = pos[:, 1]
    pad = FPS_R * FPS_C - N
    px = jnp.pad(posx, (0, pad)).reshape(FPS_R, FPS_C)
    py = jnp.pad(posy, (0, pad)).reshape(FPS_R, FPS_C)
    idx = _fps(px, py, posx, posy)

    table = jnp.concatenate(
        [h, pos, jnp.zeros((N, GATHER_D - 34), jnp.float32)], axis=1)
    idx_pad = jnp.concatenate(
        [idx, jnp.zeros((NS_PAD - NS_SAMPLES,), jnp.int32)])
    g = _sc_gather(table, idx_pad)

    xd = g[:NS_SAMPLES, :32]
    pdx = g[:NS_SAMPLES, 32].reshape(1, NS_SAMPLES)
    pdy = g[:NS_SAMPLES, 33].reshape(1, NS_SAMPLES)
    qx = posx[:, None]
    qy = posy[:, None]
    xi = _knn(qx, qy, pdx, pdy, xd)

    out = _mlp(xi, x2, W1[:32, :], W1[32:33, :], b1[None, :], g1[None, :],
               be1[None, :], W2, b2[None, :], g2[None, :], be2[None, :],
               W3, b3[None, :], g3[None, :], be3[None, :])
    return out


# speedup vs baseline: 21.2542x; 1.0030x over previous
import functools

import jax
import jax.numpy as jnp
from jax import lax
from jax.experimental import pallas as pl
from jax.experimental.pallas import tpu as pltpu
from jax.experimental.pallas import tpu_sc as plsc

N = 10000
E = 320000
KS = 5
NS_SAMPLES = 5000
KNN = 3

NC = 2
NSUB = 16
NW = NC * NSUB
LANES = 16

ROWW = 32
CNT_COL = 25
EPW = E // NW
EBATCH = 80
NGRP = EBATCH // LANES

GATHER_D = 48
NS_PAD = 5120
ROWS_PW = NS_PAD // NW

FPS_R = 80
FPS_C = 128

QBLK = 400


def _conv_scatter_body(x_hbm, src_hbm, dst_hbm, ea_hbm, zero_hbm, out_hbm,
                       x_v, src_v, dst_v, ea_v, vals_v, idx_v, b_sh):
    c = lax.axis_index("c")
    s = lax.axis_index("s")
    wid = c * NSUB + s
    ebase = wid * EPW

    zchunk = N // NSUB
    pltpu.sync_copy(zero_hbm.at[pl.ds(0, zchunk)], b_sh.at[pl.ds(s * zchunk, zchunk)])

    pltpu.sync_copy(x_hbm, x_v)
    pltpu.sync_copy(src_hbm.at[pl.ds(ebase, EPW)], src_v)
    pltpu.sync_copy(dst_hbm.at[pl.ds(ebase, EPW)], dst_v)
    pltpu.sync_copy(ea_hbm.at[pl.ds(2 * ebase, 2 * EPW)], ea_v)
    plsc.subcore_barrier()

    lanes = lax.iota(jnp.int32, LANES)
    ones16 = jnp.full((LANES,), 1.0, jnp.float32)
    zeros16 = jnp.zeros((LANES,), jnp.float32)

    @pl.loop(0, EPW // EBATCH)
    def _batch(b):
        @pl.loop(0, EBATCH)
        def _z(r):
            vals_v[r, pl.ds(0, LANES)] = zeros16
            vals_v[r, pl.ds(LANES, LANES)] = zeros16

        @pl.loop(0, NGRP)
        def _grp(g):
            off = b * EBATCH + g * LANES
            src16 = src_v[pl.ds(off, LANES)]
            dst16 = dst_v[pl.ds(off, LANES)]
            eidx = (off + lanes) * 2
            u = plsc.load_gather(ea_v, [eidx])
            v = plsc.load_gather(ea_v, [eidx + 1])
            xs = plsc.load_gather(x_v, [src16])
            vu = u * (KS - 1.0)
            vv = v * (KS - 1.0)
            bui = jnp.clip(vu.astype(jnp.int32), 0, KS - 2)
            bvi = jnp.clip(vv.astype(jnp.int32), 0, KS - 2)
            bu = bui.astype(jnp.float32)
            bv = bvi.astype(jnp.float32)
            fu = vu - bu
            fv = vv - bv
            row = g * LANES + lanes
            for i0 in (0, 1):
                for i1 in (0, 1):
                    w0 = fu if i0 else (1.0 - fu)
                    w1 = fv if i1 else (1.0 - fv)
                    val = w0 * w1 * xs
                    kk = (bui + i0) + KS * (bvi + i1)
                    plsc.addupdate_scatter(vals_v, [row, kk], val)
            plsc.addupdate_scatter(vals_v, [row, jnp.full((LANES,), CNT_COL, jnp.int32)], ones16)
            idx_v[pl.ds(g * LANES, LANES)] = dst16

        pltpu.sync_copy(vals_v, b_sh.at[idx_v], add=True)

    plsc.subcore_barrier()
    pltpu.sync_copy(b_sh.at[pl.ds(s * zchunk, zchunk)],
                    out_hbm.at[c, pl.ds(s * zchunk, zchunk)])


def _conv_scatter(x, src, dst, edge_attr_flat, zeros_rows):
    mesh = plsc.VectorSubcoreMesh(core_axis_name="c", subcore_axis_name="s")
    kfn = pl.kernel(
        _conv_scatter_body,
        out_type=jax.ShapeDtypeStruct((NC, N, ROWW), jnp.float32),
        mesh=mesh,
        scratch_types=[
            pltpu.VMEM((N,), jnp.float32),
            pltpu.VMEM((EPW,), jnp.int32),
            pltpu.VMEM((EPW,), jnp.int32),
            pltpu.VMEM((2 * EPW,), jnp.float32),
            pltpu.VMEM((EBATCH, ROWW), jnp.float32),
            pltpu.VMEM((EBATCH,), jnp.int32),
            pltpu.VMEM_SHARED((N, ROWW), jnp.float32),
        ],
        compiler_params=pltpu.CompilerParams(
            use_tc_tiling_on_sc=False, needs_layout_passes=False),
    )
    return kfn(x, src, dst, edge_attr_flat, zeros_rows)


def _combine_body(bext_ref, x_ref, wsp_ref, wroot_ref, bconv_ref, h_ref):
    b = bext_ref[0] + bext_ref[1]
    basis = b[:, :KS * KS]
    cnt = b[:, CNT_COL:CNT_COL + 1]
    num = jnp.dot(basis, wsp_ref[...], preferred_element_type=jnp.float32,
                  precision=lax.Precision.HIGHEST)
    aggr = num / jnp.maximum(cnt, 1.0)
    h = aggr + x_ref[...] * wroot_ref[...] + bconv_ref[...]
    h_ref[...] = jnp.where(h > 0.0, h, jnp.exp(h) - 1.0)


def _combine(bext, x2, wsp25, wroot, bconv2):
    return pl.pallas_call(
        _combine_body,
        out_shape=jax.ShapeDtypeStruct((N, 32), jnp.float32),
    )(bext, x2, wsp25, wroot, bconv2)


def _fps_body(px_ref, py_ref, pxs_ref, pys_ref, idx_ref):
    px = px_ref[...]
    py = py_ref[...]
    lin = lax.broadcasted_iota(jnp.int32, (FPS_R, FPS_C), 0) * FPS_C + \
        lax.broadcasted_iota(jnp.int32, (FPS_R, FPS_C), 1)
    valid = lin < N
    sx = pxs_ref[0]
    sy = pys_ref[0]
    dx = px - sx
    dy = py - sy
    d0 = dx * dx + dy * dy
    dists0 = jnp.where(valid, d0, -1.0)
    idx_ref[0] = jnp.int32(0)

    def body(i, dists):
        sel = jnp.argmax(dists).astype(jnp.int32)
        nx = pxs_ref[sel]
        ny = pys_ref[sel]
        ddx = px - nx
        ddy = py - ny
        d = ddx * ddx + ddy * ddy
        idx_ref[i] = sel
        return jnp.minimum(dists, d)

    lax.fori_loop(1, NS_SAMPLES, body, dists0)


def _fps(px, py, pxs, pys):
    return pl.pallas_call(
        _fps_body,
        in_specs=[
            pl.BlockSpec(memory_space=pltpu.VMEM),
            pl.BlockSpec(memory_space=pltpu.VMEM),
            pl.BlockSpec(memory_space=pltpu.SMEM),
            pl.BlockSpec(memory_space=pltpu.SMEM),
        ],
        out_shape=jax.ShapeDtypeStruct((NS_SAMPLES,), jnp.int32),
        out_specs=pl.BlockSpec(memory_space=pltpu.SMEM),
    )(px, py, pxs, pys)


def _gather_body(tab_hbm, idx_hbm, out_hbm, idx_v, rows_v, sem):
    c = lax.axis_index("c")
    s = lax.axis_index("s")
    wid = s * NC + c
    base = wid * ROWS_PW
    pltpu.sync_copy(idx_hbm.at[pl.ds(base, ROWS_PW)], idx_v)
    pltpu.async_copy(tab_hbm.at[idx_v], rows_v, sem).wait()
    pltpu.sync_copy(rows_v, out_hbm.at[pl.ds(base, ROWS_PW)])


def _sc_gather(table, idx_pad):
    mesh = plsc.VectorSubcoreMesh(core_axis_name="c", subcore_axis_name="s")
    kfn = pl.kernel(
        _gather_body,
        out_type=jax.ShapeDtypeStruct((NS_PAD, GATHER_D), jnp.float32),
        mesh=mesh,
        scratch_types=[
            pltpu.VMEM((ROWS_PW,), jnp.int32),
            pltpu.VMEM((ROWS_PW, GATHER_D), jnp.float32),
            pltpu.SemaphoreType.DMA,
        ],
        compiler_params=pltpu.CompilerParams(
            use_tc_tiling_on_sc=False, needs_layout_passes=False),
    )
    return kfn(table, idx_pad)


def _knn_body(qx_ref, qy_ref, pdx_ref, pdy_ref, xd_ref, xi_ref):
    qx = qx_ref[...]
    qy = qy_ref[...]
    pdx = pdx_ref[...]
    pdy = pdy_ref[...]
    nd2 = pdx * pdx + pdy * pdy
    q2 = qx * qx + qy * qy
    qxb = qx.astype(jnp.bfloat16).astype(jnp.float32)
    qyb = qy.astype(jnp.bfloat16).astype(jnp.float32)
    pdxb = pdx.astype(jnp.bfloat16).astype(jnp.float32)
    pdyb = pdy.astype(jnp.bfloat16).astype(jnp.float32)
    mm = qxb * pdxb + qyb * pdyb
    d2 = (q2 + nd2) - 2.0 * mm
    citer = lax.broadcasted_iota(jnp.int32, (QBLK, NS_SAMPLES), 1)
    inf = jnp.float32(jnp.inf)
    S = jnp.zeros((QBLK, NS_SAMPLES), jnp.float32)
    ws = jnp.zeros((QBLK, 1), jnp.float32)
    for _ in range(KNN):
        m = jnp.min(d2, axis=1, keepdims=True)
        j = jnp.min(jnp.where(d2 == m, citer, jnp.int32(2**30)), axis=1,
                    keepdims=True)
        w = 1.0 / jnp.maximum(m, 1e-16)
        oh = citer == j
        S = S + jnp.where(oh, w, 0.0)
        ws = ws + w
        d2 = jnp.where(oh, inf, d2)
    xi = jnp.dot(S, xd_ref[...], preferred_element_type=jnp.float32,
                 precision=lax.Precision.HIGHEST)
    xi_ref[...] = xi / ws


def _knn(qx, qy, pdx, pdy, xd):
    grid = N // QBLK
    return pl.pallas_call(
        _knn_body,
        grid=(grid,),
        in_specs=[
            pl.BlockSpec((QBLK, 1), lambda i: (i, 0)),
            pl.BlockSpec((QBLK, 1), lambda i: (i, 0)),
            pl.BlockSpec((1, NS_SAMPLES), lambda i: (0, 0)),
            pl.BlockSpec((1, NS_SAMPLES), lambda i: (0, 0)),
            pl.BlockSpec((NS_SAMPLES, 32), lambda i: (0, 0)),
        ],
        out_specs=pl.BlockSpec((QBLK, 32), lambda i: (i, 0)),
        out_shape=jax.ShapeDtypeStruct((N, 32), jnp.float32),
    )(qx, qy, pdx, pdy, xd)


def _bn(h, g, beta):
    mu = jnp.mean(h, axis=0, keepdims=True)
    var = jnp.mean((h - mu) ** 2, axis=0, keepdims=True)
    return g * (h - mu) / jnp.sqrt(var + 1e-5) + beta


def _mlp_body(xi_ref, x2_ref, w1a_ref, w1b_ref, b1_ref, g1_ref, be1_ref,
              w2_ref, b2_ref, g2_ref, be2_ref, w3_ref, b3_ref, g3_ref, be3_ref,
              out_ref):
    def bdot(a, b):
        return jnp.dot(a.astype(jnp.bfloat16), b.astype(jnp.bfloat16),
                       preferred_element_type=jnp.float32)

    xi = xi_ref[...]
    x2b = x2_ref[...].astype(jnp.bfloat16).astype(jnp.float32)
    w1bb = w1b_ref[...].astype(jnp.bfloat16).astype(jnp.float32)
    h = bdot(xi, w1a_ref[...]) + x2b * w1bb + b1_ref[...]
    h = jnp.maximum(h, 0.0)
    h = _bn(h, g1_ref[...], be1_ref[...])
    h = bdot(h, w2_ref[...]) + b2_ref[...]
    h = jnp.maximum(h, 0.0)
    h = _bn(h, g2_ref[...], be2_ref[...])
    h = bdot(h, w3_ref[...]) + b3_ref[...]
    h = jnp.maximum(h, 0.0)
    h = _bn(h, g3_ref[...], be3_ref[...])
    out_ref[...] = 1.0 / (1.0 + jnp.exp(-h))


def _mlp(xi, x2, w1a, w1b, b1, g1, be1, W2, b2, g2, be2, W3, b3, g3, be3):
    args = (xi, x2, w1a, w1b, b1, g1, be1, W2, b2, g2, be2, W3, b3, g3, be3)
    return pl.pallas_call(
        _mlp_body,
        out_shape=jax.ShapeDtypeStruct((N, 1), jnp.float32),
    )(*args)


def kernel(x, pos, batch, edge_index, edge_attr, Wsp, Wroot, bconv,
           W1, b1, g1, be1, W2, b2, g2, be2, W3, b3, g3, be3):
    x = x.astype(jnp.float32)
    ea_flat = edge_attr.reshape(-1)
    zeros_rows = jnp.zeros((N // NSUB, ROWW), jnp.float32)

    bext = _conv_scatter(x, edge_index[0], edge_index[1], ea_flat, zeros_rows)

    x2 = x[:, None]
    wsp25 = Wsp[:, 0, :]
    h = _combine(bext, x2, wsp25, Wroot, bconv[None, :])

    posx = pos[:, 0]
    posy = pos[:, 1]
    pad = FPS_R * FPS_C - N
    px = jnp.pad(posx, (0, pad)).reshape(FPS_R, FPS_C)
    py = jnp.pad(posy, (0, pad)).reshape(FPS_R, FPS_C)
    idx = _fps(px, py, posx, posy)

    table = jnp.concatenate(
        [h, pos, jnp.zeros((N, GATHER_D - 34), jnp.float32)], axis=1)
    idx_pad = jnp.concatenate(
        [idx, jnp.zeros((NS_PAD - NS_SAMPLES,), jnp.int32)])
    g = _sc_gather(table, idx_pad)

    xd = g[:NS_SAMPLES, :32]
    pdx = g[:NS_SAMPLES, 32].reshape(1, NS_SAMPLES)
    pdy = g[:NS_SAMPLES, 33].reshape(1, NS_SAMPLES)
    qx = posx[:, None]
    qy = posy[:, None]
    xi = _knn(qx, qy, pdx, pdy, xd)

    out = _mlp(xi, x2, W1[:32, :], W1[32:33, :], b1[None, :], g1[None, :],
               be1[None, :], W2, b2[None, :], g2[None, :], be2[None, :],
               W3, b3[None, :], g3[None, :], be3[None, :])
    return out


# speedup vs baseline: 21.3274x; 1.0034x over previous
import functools

import jax
import jax.numpy as jnp
from jax import lax
from jax.experimental import pallas as pl
from jax.experimental.pallas import tpu as pltpu
from jax.experimental.pallas import tpu_sc as plsc

N = 10000
E = 320000
KS = 5
NS_SAMPLES = 5000
KNN = 3

NC = 2
NSUB = 16
NW = NC * NSUB
LANES = 16

ROWW = 32
CNT_COL = 25
EPW = E // NW
EBATCH = 80
NGRP = EBATCH // LANES

GATHER_D = 48
NS_PAD = 5120
ROWS_PW = NS_PAD // NW

FPS_R = 80
FPS_C = 128

QBLK = 400


def _conv_scatter_body(x_hbm, src_hbm, dst_hbm, ea_hbm, zero_hbm, out_hbm,
                       x_v, src_v, dst_v, ea_v, vals_v, idx_v, b_sh):
    c = lax.axis_index("c")
    s = lax.axis_index("s")
    wid = c * NSUB + s
    ebase = wid * EPW

    zchunk = N // NSUB
    pltpu.sync_copy(zero_hbm.at[pl.ds(0, zchunk)], b_sh.at[pl.ds(s * zchunk, zchunk)])

    pltpu.sync_copy(x_hbm, x_v)
    pltpu.sync_copy(src_hbm.at[pl.ds(ebase, EPW)], src_v)
    pltpu.sync_copy(dst_hbm.at[pl.ds(ebase, EPW)], dst_v)
    pltpu.sync_copy(ea_hbm.at[pl.ds(2 * ebase, 2 * EPW)], ea_v)
    plsc.subcore_barrier()

    lanes = lax.iota(jnp.int32, LANES)
    ones16 = jnp.full((LANES,), 1.0, jnp.float32)
    zeros16 = jnp.zeros((LANES,), jnp.float32)

    @pl.loop(0, EPW // EBATCH)
    def _batch(b):
        @pl.loop(0, EBATCH)
        def _z(r):
            vals_v[r, pl.ds(0, LANES)] = zeros16
            vals_v[r, pl.ds(LANES, LANES)] = zeros16

        @pl.loop(0, NGRP)
        def _grp(g):
            off = b * EBATCH + g * LANES
            src16 = src_v[pl.ds(off, LANES)]
            dst16 = dst_v[pl.ds(off, LANES)]
            eidx = (off + lanes) * 2
            u = plsc.load_gather(ea_v, [eidx])
            v = plsc.load_gather(ea_v, [eidx + 1])
            xs = plsc.load_gather(x_v, [src16])
            vu = u * (KS - 1.0)
            vv = v * (KS - 1.0)
            bui = jnp.clip(vu.astype(jnp.int32), 0, KS - 2)
            bvi = jnp.clip(vv.astype(jnp.int32), 0, KS - 2)
            bu = bui.astype(jnp.float32)
            bv = bvi.astype(jnp.float32)
            fu = vu - bu
            fv = vv - bv
            row = g * LANES + lanes
            for i0 in (0, 1):
                for i1 in (0, 1):
                    w0 = fu if i0 else (1.0 - fu)
                    w1 = fv if i1 else (1.0 - fv)
                    val = w0 * w1 * xs
                    kk = (bui + i0) + KS * (bvi + i1)
                    plsc.addupdate_scatter(vals_v, [row, kk], val)
            plsc.addupdate_scatter(vals_v, [row, jnp.full((LANES,), CNT_COL, jnp.int32)], ones16)
            idx_v[pl.ds(g * LANES, LANES)] = dst16

        pltpu.sync_copy(vals_v, b_sh.at[idx_v], add=True)

    plsc.subcore_barrier()
    pltpu.sync_copy(b_sh.at[pl.ds(s * zchunk, zchunk)],
                    out_hbm.at[c, pl.ds(s * zchunk, zchunk)])


def _conv_scatter(x, src, dst, edge_attr_flat, zeros_rows):
    mesh = plsc.VectorSubcoreMesh(core_axis_name="c", subcore_axis_name="s")
    kfn = pl.kernel(
        _conv_scatter_body,
        out_type=jax.ShapeDtypeStruct((NC, N, ROWW), jnp.float32),
        mesh=mesh,
        scratch_types=[
            pltpu.VMEM((N,), jnp.float32),
            pltpu.VMEM((EPW,), jnp.int32),
            pltpu.VMEM((EPW,), jnp.int32),
            pltpu.VMEM((2 * EPW,), jnp.float32),
            pltpu.VMEM((EBATCH, ROWW), jnp.float32),
            pltpu.VMEM((EBATCH,), jnp.int32),
            pltpu.VMEM_SHARED((N, ROWW), jnp.float32),
        ],
        compiler_params=pltpu.CompilerParams(
            use_tc_tiling_on_sc=False, needs_layout_passes=False),
    )
    return kfn(x, src, dst, edge_attr_flat, zeros_rows)


def _combine_body(bext_ref, x_ref, px_ref, py_ref, wsp_ref, wroot_ref,
                  bconv_ref, tab_ref):
    b = bext_ref[0] + bext_ref[1]
    basis = b[:, :KS * KS]
    cnt = b[:, CNT_COL:CNT_COL + 1]
    num = jnp.dot(basis, wsp_ref[...], preferred_element_type=jnp.float32,
                  precision=lax.Precision.HIGHEST)
    aggr = num / jnp.maximum(cnt, 1.0)
    h = aggr + x_ref[...] * wroot_ref[...] + bconv_ref[...]
    h = jnp.where(h > 0.0, h, jnp.exp(h) - 1.0)
    tab_ref[:, 0:32] = h
    tab_ref[:, 32:33] = px_ref[...]
    tab_ref[:, 33:34] = py_ref[...]
    tab_ref[:, 34:GATHER_D] = jnp.zeros((CBLK, GATHER_D - 34), jnp.float32)


CBLK = 2000


def _combine(bext, x2, px2, py2, wsp25, wroot, bconv2):
    return pl.pallas_call(
        _combine_body,
        grid=(N // CBLK,),
        in_specs=[
            pl.BlockSpec((NC, CBLK, ROWW), lambda i: (0, i, 0)),
            pl.BlockSpec((CBLK, 1), lambda i: (i, 0)),
            pl.BlockSpec((CBLK, 1), lambda i: (i, 0)),
            pl.BlockSpec((CBLK, 1), lambda i: (i, 0)),
            pl.BlockSpec((KS * KS, 32), lambda i: (0, 0)),
            pl.BlockSpec((1, 32), lambda i: (0, 0)),
            pl.BlockSpec((1, 32), lambda i: (0, 0)),
        ],
        out_specs=pl.BlockSpec((CBLK, GATHER_D), lambda i: (i, 0)),
        out_shape=jax.ShapeDtypeStruct((N, GATHER_D), jnp.float32),
    )(bext, x2, px2, py2, wsp25, wroot, bconv2)


def _fps_body(px_ref, py_ref, pxs_ref, pys_ref, idx_ref):
    px = px_ref[...]
    py = py_ref[...]
    lin = lax.broadcasted_iota(jnp.int32, (FPS_R, FPS_C), 0) * FPS_C + \
        lax.broadcasted_iota(jnp.int32, (FPS_R, FPS_C), 1)
    valid = lin < N
    sx = pxs_ref[0]
    sy = pys_ref[0]
    dx = px - sx
    dy = py - sy
    d0 = dx * dx + dy * dy
    dists0 = jnp.where(valid, d0, -1.0)
    idx_ref[0] = jnp.int32(0)

    def body(i, dists):
        sel = jnp.argmax(dists).astype(jnp.int32)
        nx = pxs_ref[sel]
        ny = pys_ref[sel]
        ddx = px - nx
        ddy = py - ny
        d = ddx * ddx + ddy * ddy
        idx_ref[i] = sel
        return jnp.minimum(dists, d)

    lax.fori_loop(1, NS_SAMPLES, body, dists0)


def _fps(px, py, pxs, pys):
    return pl.pallas_call(
        _fps_body,
        in_specs=[
            pl.BlockSpec(memory_space=pltpu.VMEM),
            pl.BlockSpec(memory_space=pltpu.VMEM),
            pl.BlockSpec(memory_space=pltpu.SMEM),
            pl.BlockSpec(memory_space=pltpu.SMEM),
        ],
        out_shape=jax.ShapeDtypeStruct((NS_SAMPLES,), jnp.int32),
        out_specs=pl.BlockSpec(memory_space=pltpu.SMEM),
    )(px, py, pxs, pys)


def _gather_body(tab_hbm, idx_hbm, out_hbm, idx_v, rows_v, sem):
    c = lax.axis_index("c")
    s = lax.axis_index("s")
    wid = s * NC + c
    base = wid * ROWS_PW
    pltpu.sync_copy(idx_hbm.at[pl.ds(base, ROWS_PW)], idx_v)
    pltpu.async_copy(tab_hbm.at[idx_v], rows_v, sem).wait()
    pltpu.sync_copy(rows_v, out_hbm.at[pl.ds(base, ROWS_PW)])


def _sc_gather(table, idx_pad):
    mesh = plsc.VectorSubcoreMesh(core_axis_name="c", subcore_axis_name="s")
    kfn = pl.kernel(
        _gather_body,
        out_type=jax.ShapeDtypeStruct((NS_PAD, GATHER_D), jnp.float32),
        mesh=mesh,
        scratch_types=[
            pltpu.VMEM((ROWS_PW,), jnp.int32),
            pltpu.VMEM((ROWS_PW, GATHER_D), jnp.float32),
            pltpu.SemaphoreType.DMA,
        ],
        compiler_params=pltpu.CompilerParams(
            use_tc_tiling_on_sc=False, needs_layout_passes=False),
    )
    return kfn(table, idx_pad)


def _knn_body(qx_ref, qy_ref, pdx_ref, pdy_ref, xd_ref, xi_ref):
    qx = qx_ref[...]
    qy = qy_ref[...]
    pdx = pdx_ref[...]
    pdy = pdy_ref[...]
    nd2 = pdx * pdx + pdy * pdy
    q2 = qx * qx + qy * qy
    qxb = qx.astype(jnp.bfloat16).astype(jnp.float32)
    qyb = qy.astype(jnp.bfloat16).astype(jnp.float32)
    pdxb = pdx.astype(jnp.bfloat16).astype(jnp.float32)
    pdyb = pdy.astype(jnp.bfloat16).astype(jnp.float32)
    mm = qxb * pdxb + qyb * pdyb
    d2 = (q2 + nd2) - 2.0 * mm
    citer = lax.broadcasted_iota(jnp.int32, (QBLK, NS_SAMPLES), 1)
    inf = jnp.float32(jnp.inf)
    S = jnp.zeros((QBLK, NS_SAMPLES), jnp.float32)
    ws = jnp.zeros((QBLK, 1), jnp.float32)
    for _ in range(KNN):
        m = jnp.min(d2, axis=1, keepdims=True)
        j = jnp.min(jnp.where(d2 == m, citer, jnp.int32(2**30)), axis=1,
                    keepdims=True)
        w = 1.0 / jnp.maximum(m, 1e-16)
        oh = citer == j
        S = S + jnp.where(oh, w, 0.0)
        ws = ws + w
        d2 = jnp.where(oh, inf, d2)
    xi = jnp.dot(S, xd_ref[...], preferred_element_type=jnp.float32,
                 precision=lax.Precision.HIGHEST)
    xi_ref[...] = xi / ws


def _knn(qx, qy, pdx, pdy, xd):
    grid = N // QBLK
    return pl.pallas_call(
        _knn_body,
        grid=(grid,),
        in_specs=[
            pl.BlockSpec((QBLK, 1), lambda i: (i, 0)),
            pl.BlockSpec((QBLK, 1), lambda i: (i, 0)),
            pl.BlockSpec((1, NS_SAMPLES), lambda i: (0, 0)),
            pl.BlockSpec((1, NS_SAMPLES), lambda i: (0, 0)),
            pl.BlockSpec((NS_SAMPLES, 32), lambda i: (0, 0)),
        ],
        out_specs=pl.BlockSpec((QBLK, 32), lambda i: (i, 0)),
        out_shape=jax.ShapeDtypeStruct((N, 32), jnp.float32),
    )(qx, qy, pdx, pdy, xd)


def _bn(h, g, beta):
    mu = jnp.mean(h, axis=0, keepdims=True)
    var = jnp.mean((h - mu) ** 2, axis=0, keepdims=True)
    return g * (h - mu) / jnp.sqrt(var + 1e-5) + beta


def _mlp_body(xi_ref, x2_ref, w1a_ref, w1b_ref, b1_ref, g1_ref, be1_ref,
              w2_ref, b2_ref, g2_ref, be2_ref, w3_ref, b3_ref, g3_ref, be3_ref,
              out_ref):
    def bdot(a, b):
        return jnp.dot(a.astype(jnp.bfloat16), b.astype(jnp.bfloat16),
                       preferred_element_type=jnp.float32)

    xi = xi_ref[...]
    x2b = x2_ref[...].astype(jnp.bfloat16).astype(jnp.float32)
    w1bb = w1b_ref[...].astype(jnp.bfloat16).astype(jnp.float32)
    h = bdot(xi, w1a_ref[...]) + x2b * w1bb + b1_ref[...]
    h = jnp.maximum(h, 0.0)
    h = _bn(h, g1_ref[...], be1_ref[...])
    h = bdot(h, w2_ref[...]) + b2_ref[...]
    h = jnp.maximum(h, 0.0)
    h = _bn(h, g2_ref[...], be2_ref[...])
    h = bdot(h, w3_ref[...]) + b3_ref[...]
    h = jnp.maximum(h, 0.0)
    h = _bn(h, g3_ref[...], be3_ref[...])
    out_ref[...] = 1.0 / (1.0 + jnp.exp(-h))


def _mlp(xi, x2, w1a, w1b, b1, g1, be1, W2, b2, g2, be2, W3, b3, g3, be3):
    args = (xi, x2, w1a, w1b, b1, g1, be1, W2, b2, g2, be2, W3, b3, g3, be3)
    return pl.pallas_call(
        _mlp_body,
        out_shape=jax.ShapeDtypeStruct((N, 1), jnp.float32),
    )(*args)


def kernel(x, pos, batch, edge_index, edge_attr, Wsp, Wroot, bconv,
           W1, b1, g1, be1, W2, b2, g2, be2, W3, b3, g3, be3):
    x = x.astype(jnp.float32)
    ea_flat = edge_attr.reshape(-1)
    zeros_rows = jnp.zeros((N // NSUB, ROWW), jnp.float32)

    bext = _conv_scatter(x, edge_index[0], edge_index[1], ea_flat, zeros_rows)

    x2 = x[:, None]
    wsp25 = Wsp[:, 0, :]
    posx = pos[:, 0]
    posy = pos[:, 1]
    table = _combine(bext, x2, posx[:, None], posy[:, None], wsp25, Wroot,
                     bconv[None, :])

    pad = FPS_R * FPS_C - N
    px = jnp.pad(posx, (0, pad)).reshape(FPS_R, FPS_C)
    py = jnp.pad(posy, (0, pad)).reshape(FPS_R, FPS_C)
    idx = _fps(px, py, posx, posy)

    idx_pad = jnp.concatenate(
        [idx, jnp.zeros((NS_PAD - NS_SAMPLES,), jnp.int32)])
    g = _sc_gather(table, idx_pad)

    xd = g[:NS_SAMPLES, :32]
    pdx = g[:NS_SAMPLES, 32].reshape(1, NS_SAMPLES)
    pdy = g[:NS_SAMPLES, 33].reshape(1, NS_SAMPLES)
    qx = posx[:, None]
    qy = posy[:, None]
    xi = _knn(qx, qy, pdx, pdy, xd)

    out = _mlp(xi, x2, W1[:32, :], W1[32:33, :], b1[None, :], g1[None, :],
               be1[None, :], W2, b2[None, :], g2[None, :], be2[None, :],
               W3, b3[None, :], g3[None, :], be3[None, :])
    return out
